# baseline (speedup 1.0000x reference)
import jax
import jax.numpy as jnp
from jax.experimental import pallas as pl
from jax.experimental.pallas import tpu as pltpu

_N_CLUSTERS = 512
_D_MODEL = 768
_N_POINTS = 576
_KMEANS_ITERS = 10


def _kmeans_kernel(x_ref, gamma_ref, beta_ref, centers_ref, soft_ref):
    x = x_ref[0]
    gamma = gamma_ref[...]
    beta = beta_ref[...]
    mu = jnp.mean(x, axis=-1, keepdims=True)
    xc = x - mu
    var = jnp.mean(xc * xc, axis=-1, keepdims=True)
    x = xc * jax.lax.rsqrt(var + 1e-5) * gamma[None, :] + beta[None, :]

    xm2 = x * -2.0
    col_ids = jax.lax.broadcasted_iota(jnp.int32, (_N_POINTS, _N_CLUSTERS), 1)
    ones_row = jnp.ones((8, _N_POINTS), dtype=jnp.float32)
    big = jnp.int32(1 << 30)

    def body(_, carry):
        d, centers = carry
        dmin = jnp.min(d, axis=-1, keepdims=True)
        eq = d == dmin
        kmin = jnp.min(jnp.where(eq, col_ids, big), axis=-1, keepdims=True)
        onehot = (col_ids == kmin).astype(jnp.float32)
        sums = jax.lax.dot_general(onehot, x, (((0,), (0,)), ((), ())),
                                   preferred_element_type=jnp.float32)
        counts_row = jax.lax.dot_general(ones_row, onehot, (((1,), (0,)), ((), ())),
                                         preferred_element_type=jnp.float32)[0:1]
        prod = jax.lax.dot_general(xm2, sums, (((1,), (1,)), ((), ())),
                                   preferred_element_type=jnp.float32)
        m_row = counts_row > 0
        r_row = jnp.where(m_row, 1.0 / jnp.maximum(counts_row, 1.0), 1.0)
        s2_row = jnp.sum(sums * sums, axis=-1)[None, :]
        c2_row = s2_row * r_row * r_row
        d_new = jnp.where(m_row, prod * r_row + c2_row, d)
        counts_col = counts_row.T
        new_centers = sums * jnp.where(counts_col > 0,
                                       1.0 / jnp.maximum(counts_col, 1.0), 1.0)
        centers = jnp.where(counts_col > 0, new_centers, centers)
        return d_new, centers

    centers0 = x[:_N_CLUSTERS]
    c2_row0 = jnp.sum(centers0 * centers0, axis=-1)[None, :]
    d0 = jax.lax.dot_general(xm2, centers0, (((1,), (1,)), ((), ())),
                             preferred_element_type=jnp.float32) + c2_row0
    d, centers = jax.lax.fori_loop(0, _KMEANS_ITERS, body, (d0, centers0))
    centers_ref[0] = centers
    m = jnp.max(-d, axis=-1, keepdims=True)
    e = jnp.exp(-d - m)
    soft_ref[0] = e / jnp.sum(e, axis=-1, keepdims=True)


def kernel(patches, gamma, beta):
    B, N, D = patches.shape
    centers, soft = pl.pallas_call(
        _kmeans_kernel,
        grid=(B,),
        in_specs=[
            pl.BlockSpec((1, N, D), lambda b: (b, 0, 0)),
            pl.BlockSpec((D,), lambda b: (0,)),
            pl.BlockSpec((D,), lambda b: (0,)),
        ],
        out_specs=[
            pl.BlockSpec((1, _N_CLUSTERS, D), lambda b: (b, 0, 0)),
            pl.BlockSpec((1, N, _N_CLUSTERS), lambda b: (b, 0, 0)),
        ],
        out_shape=[
            jax.ShapeDtypeStruct((B, _N_CLUSTERS, D), jnp.float32),
            jax.ShapeDtypeStruct((B, N, _N_CLUSTERS), jnp.float32),
        ],
        compiler_params=pltpu.CompilerParams(
            dimension_semantics=("parallel",),
        ),
    )(patches, gamma, beta)
    return (centers, soft)


# speedup vs baseline: 1.4645x; 1.4645x over previous
import jax
import jax.numpy as jnp
from jax.experimental import pallas as pl
from jax.experimental.pallas import tpu as pltpu

_N_CLUSTERS = 512
_D_MODEL = 768
_N_POINTS = 576
_KMEANS_ITERS = 10


def _kmeans_kernel(x_ref, gamma_ref, beta_ref, centers_ref, soft_ref):
    x = x_ref[0]
    gamma = gamma_ref[...]
    beta = beta_ref[...]
    mu = jnp.mean(x, axis=-1, keepdims=True)
    xc = x - mu
    var = jnp.mean(xc * xc, axis=-1, keepdims=True)
    x = xc * jax.lax.rsqrt(var + 1e-5) * gamma[None, :] + beta[None, :]

    xm2 = x * -2.0
    col_ids = jax.lax.broadcasted_iota(jnp.int32, (_N_POINTS, _N_CLUSTERS), 1)
    ones_row = jnp.ones((8, _N_POINTS), dtype=jnp.float32)
    big = jnp.int32(1 << 30)

    def dists(centers):
        c2 = jnp.sum(centers * centers, axis=-1)
        prod = jax.lax.dot_general(xm2, centers, (((1,), (1,)), ((), ())),
                                   preferred_element_type=jnp.float32)
        return prod + c2[None, :]

    def body(_, centers):
        d = dists(centers)
        dmin = jnp.min(d, axis=-1, keepdims=True)
        onehot = (d == dmin).astype(jnp.float32)
        sums = jax.lax.dot_general(onehot, x, (((0,), (0,)), ((), ())),
                                   preferred_element_type=jnp.float32)
        counts = jax.lax.dot_general(ones_row, onehot, (((1,), (0,)), ((), ())),
                                     preferred_element_type=jnp.float32)[0]
        recip = 1.0 / jnp.maximum(counts, 1.0)
        new_centers = sums * recip[:, None]
        return jnp.where(counts[:, None] > 0, new_centers, centers)

    centers = jax.lax.fori_loop(0, _KMEANS_ITERS, body, x[:_N_CLUSTERS])
    d = dists(centers)
    centers_ref[0] = centers
    m = jnp.max(-d, axis=-1, keepdims=True)
    e = jnp.exp(-d - m)
    soft_ref[0] = e / jnp.sum(e, axis=-1, keepdims=True)


def kernel(patches, gamma, beta):
    B, N, D = patches.shape
    centers, soft = pl.pallas_call(
        _kmeans_kernel,
        grid=(B,),
        in_specs=[
            pl.BlockSpec((1, N, D), lambda b: (b, 0, 0)),
            pl.BlockSpec((D,), lambda b: (0,)),
            pl.BlockSpec((D,), lambda b: (0,)),
        ],
        out_specs=[
            pl.BlockSpec((1, _N_CLUSTERS, D), lambda b: (b, 0, 0)),
            pl.BlockSpec((1, N, _N_CLUSTERS), lambda b: (b, 0, 0)),
        ],
        out_shape=[
            jax.ShapeDtypeStruct((B, _N_CLUSTERS, D), jnp.float32),
            jax.ShapeDtypeStruct((B, N, _N_CLUSTERS), jnp.float32),
        ],
        compiler_params=pltpu.CompilerParams(
            dimension_semantics=("parallel",),
        ),
    )(patches, gamma, beta)
    return (centers, soft)


# speedup vs baseline: 1.5407x; 1.0521x over previous
import jax
import jax.numpy as jnp
from jax.experimental import pallas as pl
from jax.experimental.pallas import tpu as pltpu

_N_CLUSTERS = 512
_D_MODEL = 768
_N_POINTS = 576
_KMEANS_ITERS = 10


def _kmeans_kernel(x_ref, gamma_ref, beta_ref, centers_ref, soft_ref):
    x = x_ref[0]
    gamma = gamma_ref[...]
    beta = beta_ref[...]
    mu = jnp.mean(x, axis=-1, keepdims=True)
    xc = x - mu
    var = jnp.mean(xc * xc, axis=-1, keepdims=True)
    x = xc * jax.lax.rsqrt(var + 1e-5) * gamma[None, :] + beta[None, :]

    xm2 = x * -2.0
    col_ids = jax.lax.broadcasted_iota(jnp.int32, (_N_POINTS, _N_CLUSTERS), 1)
    ones_row = jnp.ones((8, _N_POINTS), dtype=jnp.float32)
    big = jnp.int32(1 << 30)

    def dists(centers):
        c2 = jnp.sum(centers * centers, axis=-1)
        prod = jax.lax.dot_general(xm2, centers, (((1,), (1,)), ((), ())),
                                   preferred_element_type=jnp.float32)
        return prod + c2[None, :]

    def body(_, centers):
        d = dists(centers)
        dmin = jnp.min(d, axis=-1, keepdims=True)
        onehot = (d == dmin).astype(jnp.float32)
        sums = jax.lax.dot_general(onehot, x, (((0,), (0,)), ((), ())),
                                   preferred_element_type=jnp.float32)
        counts = jax.lax.dot_general(ones_row, onehot, (((1,), (0,)), ((), ())),
                                     preferred_element_type=jnp.float32)[0]
        recip = 1.0 / jnp.maximum(counts, 1.0)
        new_centers = sums * recip[:, None]
        return jnp.where(counts[:, None] > 0, new_centers, centers)

    def body2(i, centers):
        return body(i, body(i, centers))

    centers = jax.lax.fori_loop(0, _KMEANS_ITERS // 2, body2, x[:_N_CLUSTERS])
    d = dists(centers)
    centers_ref[0] = centers
    m = jnp.max(-d, axis=-1, keepdims=True)
    e = jnp.exp(-d - m)
    soft_ref[0] = e / jnp.sum(e, axis=-1, keepdims=True)


def kernel(patches, gamma, beta):
    B, N, D = patches.shape
    centers, soft = pl.pallas_call(
        _kmeans_kernel,
        grid=(B,),
        in_specs=[
            pl.BlockSpec((1, N, D), lambda b: (b, 0, 0)),
            pl.BlockSpec((D,), lambda b: (0,)),
            pl.BlockSpec((D,), lambda b: (0,)),
        ],
        out_specs=[
            pl.BlockSpec((1, _N_CLUSTERS, D), lambda b: (b, 0, 0)),
            pl.BlockSpec((1, N, _N_CLUSTERS), lambda b: (b, 0, 0)),
        ],
        out_shape=[
            jax.ShapeDtypeStruct((B, _N_CLUSTERS, D), jnp.float32),
            jax.ShapeDtypeStruct((B, N, _N_CLUSTERS), jnp.float32),
        ],
        compiler_params=pltpu.CompilerParams(
            dimension_semantics=("parallel",),
        ),
    )(patches, gamma, beta)
    return (centers, soft)


# speedup vs baseline: 1.7772x; 1.1535x over previous
import jax
import jax.numpy as jnp
from jax.experimental import pallas as pl
from jax.experimental.pallas import tpu as pltpu

_N_CLUSTERS = 512
_D_MODEL = 768
_N_POINTS = 576
_KMEANS_ITERS = 10


def _kmeans_kernel(x_ref, gamma_ref, beta_ref, centers_ref, soft_ref):
    x = x_ref[0]
    gamma = gamma_ref[...]
    beta = beta_ref[...]
    mu = jnp.mean(x, axis=-1, keepdims=True)
    xc = x - mu
    var = jnp.mean(xc * xc, axis=-1, keepdims=True)
    x = xc * jax.lax.rsqrt(var + 1e-5) * gamma[None, :] + beta[None, :]

    xm2 = x * -2.0
    col_ids = jax.lax.broadcasted_iota(jnp.int32, (_N_POINTS, _N_CLUSTERS), 1)
    ones_row = jnp.ones((8, _N_POINTS), dtype=jnp.float32)
    big = jnp.int32(1 << 30)

    def dists(centers):
        c2 = jnp.sum(centers * centers, axis=-1)
        prod = jax.lax.dot_general(xm2, centers, (((1,), (1,)), ((), ())),
                                   preferred_element_type=jnp.float32)
        return prod + c2[None, :]

    def body(_, centers):
        d = dists(centers)
        dmin = jnp.min(d, axis=-1, keepdims=True)
        onehot = (d == dmin).astype(jnp.float32)
        sums = jax.lax.dot_general(onehot, x, (((0,), (0,)), ((), ())),
                                   preferred_element_type=jnp.float32)
        counts = jax.lax.dot_general(ones_row, onehot, (((1,), (0,)), ((), ())),
                                     preferred_element_type=jnp.float32)[0]
        recip = 1.0 / jnp.maximum(counts, 1.0)
        new_centers = sums * recip[:, None]
        return jnp.where(counts[:, None] > 0, new_centers, centers)

    centers = x[:_N_CLUSTERS]
    for _ in range(_KMEANS_ITERS):
        centers = body(0, centers)
    d = dists(centers)
    centers_ref[0] = centers
    m = jnp.max(-d, axis=-1, keepdims=True)
    e = jnp.exp(-d - m)
    soft_ref[0] = e / jnp.sum(e, axis=-1, keepdims=True)


def kernel(patches, gamma, beta):
    B, N, D = patches.shape
    centers, soft = pl.pallas_call(
        _kmeans_kernel,
        grid=(B,),
        in_specs=[
            pl.BlockSpec((1, N, D), lambda b: (b, 0, 0)),
            pl.BlockSpec((D,), lambda b: (0,)),
            pl.BlockSpec((D,), lambda b: (0,)),
        ],
        out_specs=[
            pl.BlockSpec((1, _N_CLUSTERS, D), lambda b: (b, 0, 0)),
            pl.BlockSpec((1, N, _N_CLUSTERS), lambda b: (b, 0, 0)),
        ],
        out_shape=[
            jax.ShapeDtypeStruct((B, _N_CLUSTERS, D), jnp.float32),
            jax.ShapeDtypeStruct((B, N, _N_CLUSTERS), jnp.float32),
        ],
        compiler_params=pltpu.CompilerParams(
            dimension_semantics=("parallel",),
        ),
    )(patches, gamma, beta)
    return (centers, soft)
